# Initial kernel scaffold; baseline (speedup 1.0000x reference)
import jax
import jax.numpy as jnp
from jax.experimental import pallas as pl


def kernel(x, edge_index, edge_attr, batch, W1a, b1a, W1b, b1b, root1, bias1, gamma1, beta1, W2a, b2a, W2b, b2b, root2, bias2, gamma2, beta2, Wc1, bc1, Wc2, bc2):
    raise NotImplementedError("write your pallas kernel here")


# speedup vs baseline: 2.0591x; 2.0591x over previous
import functools

import jax
import jax.numpy as jnp
from jax import lax
from jax.experimental import pallas as pl
from jax.experimental.pallas import tpu as pltpu
from jax.experimental.pallas import tpu_sc as plsc

_N = 10000
_E = 20000
_NODE_IN = 16
_EDGE_IN = 4
_HID = 32
_G = 8

_NW = 32
_EP = 20480
_EW = _EP // _NW
_KC = _EW // 128
_NP = 10240
_NSUB = _NP // 16
_MW = 48
_EB = 1024


def _sc_mesh():
    return plsc.VectorSubcoreMesh(core_axis_name="c", subcore_axis_name="s")


def _gather_body(table_hbm, idx3_hbm, out_hbm, idx_v, rows_v, sem):
    cid = lax.axis_index("c")
    sid = lax.axis_index("s")
    wid = sid * 2 + cid
    pltpu.sync_copy(idx3_hbm.at[wid], idx_v)
    cps = [
        pltpu.async_copy(table_hbm.at[idx_v.at[j]],
                         rows_v.at[pl.ds(j * 128, 128)], sem)
        for j in range(_KC)
    ]
    for c in cps:
        c.wait()
    pltpu.sync_copy(rows_v, out_hbm.at[pl.ds(wid * _EW, _EW)])


def _sc_gather(table, idx3, ncols):
    fn = pl.kernel(
        _gather_body,
        mesh=_sc_mesh(),
        out_type=jax.ShapeDtypeStruct((_EP, ncols), jnp.float32),
        scratch_types=[
            pltpu.VMEM((_KC, 128), jnp.int32),
            pltpu.VMEM((_EW, ncols), jnp.float32),
            pltpu.SemaphoreType.DMA,
        ],
        compiler_params=pltpu.CompilerParams(use_tc_tiling_on_sc=False),
    )
    return fn(table, idx3)


def _scatter_body(msg_hbm, idx3_hbm, zero_hbm, out_hbm, idx_v, msg_v, acc_sh):
    cid = lax.axis_index("c")
    sid = lax.axis_index("s")
    wid = sid * 2 + cid
    pltpu.sync_copy(idx3_hbm.at[wid], idx_v)
    pltpu.sync_copy(msg_hbm.at[pl.ds(wid * _EW, _EW)], msg_v)
    pltpu.sync_copy(zero_hbm.at[pl.ds(sid * _NSUB, _NSUB)],
                    acc_sh.at[pl.ds(sid * _NSUB, _NSUB)])
    plsc.subcore_barrier()
    for j in range(_KC):
        pltpu.sync_copy(msg_v.at[pl.ds(j * 128, 128)],
                        acc_sh.at[idx_v.at[j]], add=True)
    plsc.subcore_barrier()
    off = cid * _NP + sid * _NSUB
    pltpu.sync_copy(acc_sh.at[pl.ds(sid * _NSUB, _NSUB)],
                    out_hbm.at[pl.ds(off, _NSUB)])


def _sc_scatter_add(msg, idx3, zero48):
    fn = pl.kernel(
        _scatter_body,
        mesh=_sc_mesh(),
        out_type=jax.ShapeDtypeStruct((2 * _NP, _MW), jnp.float32),
        scratch_types=[
            pltpu.VMEM((_KC, 128), jnp.int32),
            pltpu.VMEM((_EW, _MW), jnp.float32),
            pltpu.VMEM_SHARED((_NP, _MW), jnp.float32),
        ],
        compiler_params=pltpu.CompilerParams(use_tc_tiling_on_sc=False),
    )
    return fn(msg, idx3, zero48)


def _dense_body(in_ch, ea_ref, wa_ref, ba_ref, wb_ref, bb_ref, hs_ref, out_ref):
    b = pl.program_id(0)
    eb = ea_ref.shape[0]
    h = wa_ref.shape[1]
    f32 = jnp.float32
    a = jnp.dot(ea_ref[...], wa_ref[...], preferred_element_type=f32) + ba_ref[...]
    r = jnp.maximum(a, 0.0)
    w = jnp.dot(r, wb_ref[...], preferred_element_type=f32) + bb_ref[...]
    kcol = lax.broadcasted_iota(jnp.int32, (in_ch, h), 1) // _HID
    irow = lax.broadcasted_iota(jnp.int32, (in_ch, h), 0)
    rmat = (kcol == irow).astype(f32)
    hmat = jnp.dot(hs_ref[...], rmat, preferred_element_type=f32)
    krow = lax.broadcasted_iota(jnp.int32, (h, _HID), 0) % _HID
    ocol = lax.broadcasted_iota(jnp.int32, (h, _HID), 1)
    smat = (krow == ocol).astype(f32)
    msg = jnp.dot(hmat * w, smat, preferred_element_type=f32)
    rows = b * eb + lax.broadcasted_iota(jnp.int32, (eb, 1), 0)
    valid = (rows < _E).astype(f32)
    out_ref[...] = jnp.concatenate(
        [msg * valid, valid, jnp.zeros((eb, _MW - _HID - 1), f32)], axis=1)


def _tc_dense(ea_p, wa, ba, wb, bb, hs, in_ch):
    h = wa.shape[1]
    grid = (_EP // _EB,)
    return pl.pallas_call(
        functools.partial(_dense_body, in_ch),
        grid=grid,
        in_specs=[
            pl.BlockSpec((_EB, _EDGE_IN), lambda i: (i, 0)),
            pl.BlockSpec((_EDGE_IN, h), lambda i: (0, 0)),
            pl.BlockSpec((1, h), lambda i: (0, 0)),
            pl.BlockSpec((h, h), lambda i: (0, 0)),
            pl.BlockSpec((1, h), lambda i: (0, 0)),
            pl.BlockSpec((_EB, in_ch), lambda i: (i, 0)),
        ],
        out_specs=pl.BlockSpec((_EB, _MW), lambda i: (i, 0)),
        out_shape=jax.ShapeDtypeStruct((_EP, _MW), jnp.float32),
    )(ea_p, wa, ba.reshape(1, h), wb, bb.reshape(1, h), hs)


def _agg_bn_relu(acc, hprev, root, bias, gamma, beta):
    f32 = jnp.float32
    sp = acc[0] + acc[1]
    s = sp[:, :_HID]
    cnt = sp[:, _HID:_HID + 1]
    inv = 1.0 / jnp.maximum(cnt, 1.0)
    pre = s * inv + jnp.dot(hprev, root, preferred_element_type=f32) + bias
    rows = lax.broadcasted_iota(jnp.int32, (pre.shape[0], 1), 0)
    mask = rows < _N
    pre_m = jnp.where(mask, pre, 0.0)
    mu = jnp.sum(pre_m, axis=0, keepdims=True) * (1.0 / _N)
    var = jnp.sum(jnp.where(mask, (pre - mu) ** 2, 0.0), axis=0,
                  keepdims=True) * (1.0 / _N)
    hn = gamma * (pre - mu) * lax.rsqrt(var + 1e-5) + beta
    return jnp.where(mask, jnp.maximum(hn, 0.0), 0.0)


def _combine_body(acc_ref, x_ref, root_ref, bias_ref, gamma_ref, beta_ref,
                  out_ref):
    out_ref[...] = _agg_bn_relu(acc_ref[...], x_ref[...], root_ref[...],
                                bias_ref[...], gamma_ref[...], beta_ref[...])


def _tc_combine(acc, hprev, root, bias, gamma, beta):
    return pl.pallas_call(
        _combine_body,
        out_shape=jax.ShapeDtypeStruct((_NP, _HID), jnp.float32),
    )(acc, hprev, root, bias.reshape(1, -1), gamma.reshape(1, -1),
      beta.reshape(1, -1))


def _final_body(acc_ref, h1_ref, root_ref, bias_ref, gamma_ref, beta_ref,
                batch_ref, wc1_ref, bc1_ref, wc2_ref, bc2_ref, out_ref):
    f32 = jnp.float32
    h2 = _agg_bn_relu(acc_ref[...], h1_ref[...], root_ref[...], bias_ref[...],
                      gamma_ref[...], beta_ref[...])
    gid = lax.broadcasted_iota(jnp.int32, (_NP, _G), 1)
    onehot = (batch_ref[...] == gid).astype(f32)
    seg = lax.dot_general(onehot, h2, (((0,), (0,)), ((), ())),
                          preferred_element_type=f32)
    ones = jnp.ones((_NP, 1), f32)
    cg = lax.dot_general(onehot, ones, (((0,), (0,)), ((), ())),
                         preferred_element_type=f32)
    pooled = seg * (1.0 / jnp.maximum(cg, 1.0))
    hc = jnp.maximum(
        jnp.dot(pooled, wc1_ref[...], preferred_element_type=f32)
        + bc1_ref[...], 0.0)
    out_ref[...] = (jnp.dot(hc, wc2_ref[...], preferred_element_type=f32)
                    + bc2_ref[...])


def _tc_final(acc, h1, root, bias, gamma, beta, batch2, wc1, bc1, wc2, bc2):
    return pl.pallas_call(
        _final_body,
        out_shape=jax.ShapeDtypeStruct((_G, 1), jnp.float32),
    )(acc, h1, root, bias.reshape(1, -1), gamma.reshape(1, -1),
      beta.reshape(1, -1), batch2, wc1, bc1.reshape(1, -1), wc2,
      bc2.reshape(1, 1))


def kernel(x, edge_index, edge_attr, batch, W1a, b1a, W1b, b1b, root1, bias1,
           gamma1, beta1, W2a, b2a, W2b, b2b, root2, bias2, gamma2, beta2,
           Wc1, bc1, Wc2, bc2):
    f32 = jnp.float32
    i32 = jnp.int32
    pe = _EP - _E
    pn = _NP - _N
    src3 = jnp.concatenate([edge_index[0], jnp.zeros((pe,), i32)]
                           ).reshape(_NW, _KC, 128)
    dst3 = jnp.concatenate([edge_index[1], jnp.zeros((pe,), i32)]
                           ).reshape(_NW, _KC, 128)
    ea_p = jnp.concatenate([edge_attr, jnp.zeros((pe, _EDGE_IN), f32)], axis=0)
    x_p = jnp.concatenate([x, jnp.zeros((pn, _NODE_IN), f32)], axis=0)
    batch2 = jnp.concatenate([batch, jnp.full((pn,), _G, i32)]).reshape(_NP, 1)
    zero48 = jnp.zeros((_NP, _MW), f32)

    xs = _sc_gather(x, src3, _NODE_IN)
    m1 = _tc_dense(ea_p, W1a, b1a, W1b, b1b, xs, _NODE_IN)
    a1 = _sc_scatter_add(m1, dst3, zero48)
    h1 = _tc_combine(a1.reshape(2, _NP, _MW), x_p, root1, bias1, gamma1, beta1)

    h1s = _sc_gather(h1, src3, _HID)
    m2 = _tc_dense(ea_p, W2a, b2a, W2b, b2b, h1s, _HID)
    a2 = _sc_scatter_add(m2, dst3, zero48)

    return _tc_final(a2.reshape(2, _NP, _MW), h1, root2, bias2, gamma2, beta2,
                     batch2, Wc1, bc1, Wc2, bc2)


# speedup vs baseline: 2.0597x; 1.0003x over previous
import functools

import jax
import jax.numpy as jnp
from jax import lax
from jax.experimental import pallas as pl
from jax.experimental.pallas import tpu as pltpu
from jax.experimental.pallas import tpu_sc as plsc

_N = 10000
_E = 20000
_NODE_IN = 16
_EDGE_IN = 4
_HID = 32
_G = 8

_NW = 32
_EP = 20480
_EW = _EP // _NW
_KC = _EW // 128
_NP = 10240
_NSUB = _NP // 16
_MW = 48
_EB = 1024


def _sc_mesh():
    return plsc.VectorSubcoreMesh(core_axis_name="c", subcore_axis_name="s")


def _gather_body(table_hbm, idx3_hbm, out_hbm, idx_v, rows_v, sem):
    cid = lax.axis_index("c")
    sid = lax.axis_index("s")
    wid = sid * 2 + cid
    pltpu.sync_copy(idx3_hbm.at[wid], idx_v)
    cps = [
        pltpu.async_copy(table_hbm.at[idx_v.at[j]],
                         rows_v.at[pl.ds(j * 128, 128)], sem)
        for j in range(_KC)
    ]
    for c in cps:
        c.wait()
    pltpu.sync_copy(rows_v, out_hbm.at[pl.ds(wid * _EW, _EW)])


def _sc_gather(table, idx3, ncols):
    fn = pl.kernel(
        _gather_body,
        mesh=_sc_mesh(),
        out_type=jax.ShapeDtypeStruct((_EP, ncols), jnp.float32),
        scratch_types=[
            pltpu.VMEM((_KC, 128), jnp.int32),
            pltpu.VMEM((_EW, ncols), jnp.float32),
            pltpu.SemaphoreType.DMA,
        ],
        compiler_params=pltpu.CompilerParams(use_tc_tiling_on_sc=False),
    )
    return fn(table, idx3)


def _scatter_body(msg_hbm, idx3_hbm, zero_hbm, out_hbm, idx_v, msg_v, acc_sh):
    cid = lax.axis_index("c")
    sid = lax.axis_index("s")
    wid = sid * 2 + cid
    pltpu.sync_copy(idx3_hbm.at[wid], idx_v)
    pltpu.sync_copy(msg_hbm.at[pl.ds(wid * _EW, _EW)], msg_v)
    pltpu.sync_copy(zero_hbm.at[pl.ds(sid * _NSUB, _NSUB)],
                    acc_sh.at[pl.ds(sid * _NSUB, _NSUB)])
    plsc.subcore_barrier()
    for j in range(_KC):
        pltpu.sync_copy(msg_v.at[pl.ds(j * 128, 128)],
                        acc_sh.at[idx_v.at[j]], add=True)
    plsc.subcore_barrier()
    off = cid * _NP + sid * _NSUB
    pltpu.sync_copy(acc_sh.at[pl.ds(sid * _NSUB, _NSUB)],
                    out_hbm.at[pl.ds(off, _NSUB)])


def _sc_scatter_add(msg, idx3, zero48):
    fn = pl.kernel(
        _scatter_body,
        mesh=_sc_mesh(),
        out_type=jax.ShapeDtypeStruct((2 * _NP, _MW), jnp.float32),
        scratch_types=[
            pltpu.VMEM((_KC, 128), jnp.int32),
            pltpu.VMEM((_EW, _MW), jnp.float32),
            pltpu.VMEM_SHARED((_NP, _MW), jnp.float32),
        ],
        compiler_params=pltpu.CompilerParams(use_tc_tiling_on_sc=False),
    )
    return fn(msg, idx3, zero48)


def _dense_body(in_ch, ea_ref, wa_ref, ba_ref, wb_ref, bb_ref, hs_ref, out_ref):
    b = pl.program_id(0)
    eb = ea_ref.shape[0]
    h = wa_ref.shape[1]
    f32 = jnp.float32
    a = jnp.dot(ea_ref[...], wa_ref[...], preferred_element_type=f32) + ba_ref[...]
    r = jnp.maximum(a, 0.0).astype(jnp.bfloat16)
    w = jnp.dot(r, wb_ref[...].astype(jnp.bfloat16),
                preferred_element_type=f32) + bb_ref[...]
    kcol = lax.broadcasted_iota(jnp.int32, (in_ch, h), 1) // _HID
    irow = lax.broadcasted_iota(jnp.int32, (in_ch, h), 0)
    rmat = (kcol == irow).astype(f32)
    hmat = jnp.dot(hs_ref[...], rmat, preferred_element_type=f32)
    krow = lax.broadcasted_iota(jnp.int32, (h, _HID), 0) % _HID
    ocol = lax.broadcasted_iota(jnp.int32, (h, _HID), 1)
    smat = (krow == ocol).astype(jnp.bfloat16)
    msg = jnp.dot((hmat * w).astype(jnp.bfloat16), smat,
                  preferred_element_type=f32)
    rows = b * eb + lax.broadcasted_iota(jnp.int32, (eb, 1), 0)
    valid = (rows < _E).astype(f32)
    out_ref[...] = jnp.concatenate(
        [msg * valid, valid, jnp.zeros((eb, _MW - _HID - 1), f32)], axis=1)


def _tc_dense(ea_p, wa, ba, wb, bb, hs, in_ch):
    h = wa.shape[1]
    grid = (_EP // _EB,)
    return pl.pallas_call(
        functools.partial(_dense_body, in_ch),
        grid=grid,
        in_specs=[
            pl.BlockSpec((_EB, _EDGE_IN), lambda i: (i, 0)),
            pl.BlockSpec((_EDGE_IN, h), lambda i: (0, 0)),
            pl.BlockSpec((1, h), lambda i: (0, 0)),
            pl.BlockSpec((h, h), lambda i: (0, 0)),
            pl.BlockSpec((1, h), lambda i: (0, 0)),
            pl.BlockSpec((_EB, in_ch), lambda i: (i, 0)),
        ],
        out_specs=pl.BlockSpec((_EB, _MW), lambda i: (i, 0)),
        out_shape=jax.ShapeDtypeStruct((_EP, _MW), jnp.float32),
    )(ea_p, wa, ba.reshape(1, h), wb, bb.reshape(1, h), hs)


def _agg_bn_relu(acc, hprev, root, bias, gamma, beta):
    f32 = jnp.float32
    sp = acc[0] + acc[1]
    s = sp[:, :_HID]
    cnt = sp[:, _HID:_HID + 1]
    inv = 1.0 / jnp.maximum(cnt, 1.0)
    pre = s * inv + jnp.dot(hprev, root, preferred_element_type=f32) + bias
    rows = lax.broadcasted_iota(jnp.int32, (pre.shape[0], 1), 0)
    mask = rows < _N
    pre_m = jnp.where(mask, pre, 0.0)
    mu = jnp.sum(pre_m, axis=0, keepdims=True) * (1.0 / _N)
    var = jnp.sum(jnp.where(mask, (pre - mu) ** 2, 0.0), axis=0,
                  keepdims=True) * (1.0 / _N)
    hn = gamma * (pre - mu) * lax.rsqrt(var + 1e-5) + beta
    return jnp.where(mask, jnp.maximum(hn, 0.0), 0.0)


def _combine_body(acc_ref, x_ref, root_ref, bias_ref, gamma_ref, beta_ref,
                  out_ref):
    out_ref[...] = _agg_bn_relu(acc_ref[...], x_ref[...], root_ref[...],
                                bias_ref[...], gamma_ref[...], beta_ref[...])


def _tc_combine(acc, hprev, root, bias, gamma, beta):
    return pl.pallas_call(
        _combine_body,
        out_shape=jax.ShapeDtypeStruct((_NP, _HID), jnp.float32),
    )(acc, hprev, root, bias.reshape(1, -1), gamma.reshape(1, -1),
      beta.reshape(1, -1))


def _final_body(acc_ref, h1_ref, root_ref, bias_ref, gamma_ref, beta_ref,
                batch_ref, wc1_ref, bc1_ref, wc2_ref, bc2_ref, out_ref):
    f32 = jnp.float32
    h2 = _agg_bn_relu(acc_ref[...], h1_ref[...], root_ref[...], bias_ref[...],
                      gamma_ref[...], beta_ref[...])
    gid = lax.broadcasted_iota(jnp.int32, (_NP, _G), 1)
    onehot = (batch_ref[...] == gid).astype(f32)
    seg = lax.dot_general(onehot, h2, (((0,), (0,)), ((), ())),
                          preferred_element_type=f32)
    ones = jnp.ones((_NP, 1), f32)
    cg = lax.dot_general(onehot, ones, (((0,), (0,)), ((), ())),
                         preferred_element_type=f32)
    pooled = seg * (1.0 / jnp.maximum(cg, 1.0))
    hc = jnp.maximum(
        jnp.dot(pooled, wc1_ref[...], preferred_element_type=f32)
        + bc1_ref[...], 0.0)
    out_ref[...] = (jnp.dot(hc, wc2_ref[...], preferred_element_type=f32)
                    + bc2_ref[...])


def _tc_final(acc, h1, root, bias, gamma, beta, batch2, wc1, bc1, wc2, bc2):
    return pl.pallas_call(
        _final_body,
        out_shape=jax.ShapeDtypeStruct((_G, 1), jnp.float32),
    )(acc, h1, root, bias.reshape(1, -1), gamma.reshape(1, -1),
      beta.reshape(1, -1), batch2, wc1, bc1.reshape(1, -1), wc2,
      bc2.reshape(1, 1))


def kernel(x, edge_index, edge_attr, batch, W1a, b1a, W1b, b1b, root1, bias1,
           gamma1, beta1, W2a, b2a, W2b, b2b, root2, bias2, gamma2, beta2,
           Wc1, bc1, Wc2, bc2):
    f32 = jnp.float32
    i32 = jnp.int32
    pe = _EP - _E
    pn = _NP - _N
    src3 = jnp.concatenate([edge_index[0], jnp.zeros((pe,), i32)]
                           ).reshape(_NW, _KC, 128)
    dst3 = jnp.concatenate([edge_index[1], jnp.zeros((pe,), i32)]
                           ).reshape(_NW, _KC, 128)
    ea_p = jnp.concatenate([edge_attr, jnp.zeros((pe, _EDGE_IN), f32)], axis=0)
    x_p = jnp.concatenate([x, jnp.zeros((pn, _NODE_IN), f32)], axis=0)
    batch2 = jnp.concatenate([batch, jnp.full((pn,), _G, i32)]).reshape(_NP, 1)
    zero48 = jnp.zeros((_NP, _MW), f32)

    xs = _sc_gather(x, src3, _NODE_IN)
    m1 = _tc_dense(ea_p, W1a, b1a, W1b, b1b, xs, _NODE_IN)
    a1 = _sc_scatter_add(m1, dst3, zero48)
    h1 = _tc_combine(a1.reshape(2, _NP, _MW), x_p, root1, bias1, gamma1, beta1)

    h1s = _sc_gather(h1, src3, _HID)
    m2 = _tc_dense(ea_p, W2a, b2a, W2b, b2b, h1s, _HID)
    a2 = _sc_scatter_add(m2, dst3, zero48)

    return _tc_final(a2.reshape(2, _NP, _MW), h1, root2, bias2, gamma2, beta2,
                     batch2, Wc1, bc1, Wc2, bc2)


# speedup vs baseline: 2.1349x; 1.0365x over previous
import functools

import jax
import jax.numpy as jnp
from jax import lax
from jax.experimental import pallas as pl
from jax.experimental.pallas import tpu as pltpu
from jax.experimental.pallas import tpu_sc as plsc

_N = 10000
_E = 20000
_NODE_IN = 16
_EDGE_IN = 4
_HID = 32
_G = 8

_NW = 32
_EP = 20480
_EW = _EP // _NW
_KC = _EW // 128
_NP = 10240
_NSUB = _NP // 16
_MW = 128
_AW = 48
_EB = 1024


def _sc_mesh():
    return plsc.VectorSubcoreMesh(core_axis_name="c", subcore_axis_name="s")


def _gather_body(table_hbm, idx3_hbm, out_hbm, idx_v, rows_v, sem):
    cid = lax.axis_index("c")
    sid = lax.axis_index("s")
    wid = sid * 2 + cid
    pltpu.sync_copy(idx3_hbm.at[wid], idx_v)
    cps = [
        pltpu.async_copy(table_hbm.at[idx_v.at[j]],
                         rows_v.at[pl.ds(j * 128, 128)], sem)
        for j in range(_KC)
    ]
    for c in cps:
        c.wait()
    pltpu.sync_copy(rows_v, out_hbm.at[pl.ds(wid * _EW, _EW)])


_GATHER_FN = None


def _sc_gather(table, idx3):
    global _GATHER_FN
    if _GATHER_FN is None:
        _GATHER_FN = pl.kernel(
            _gather_body,
            mesh=_sc_mesh(),
            out_type=jax.ShapeDtypeStruct((_EP, _MW), jnp.float32),
            scratch_types=[
                pltpu.VMEM((8, 128), jnp.int32),
                pltpu.VMEM((_EW, _MW), jnp.float32),
                pltpu.SemaphoreType.DMA,
            ],
            compiler_params=pltpu.CompilerParams(use_tc_tiling_on_sc=False),
        )
    return _GATHER_FN(table, idx3)


def _scatter_body(msg_hbm, idx3_hbm, zero_hbm, out_hbm, idx_v, msg_v, acc_sh):
    cid = lax.axis_index("c")
    sid = lax.axis_index("s")
    wid = sid * 2 + cid
    pltpu.sync_copy(idx3_hbm.at[wid], idx_v)
    pltpu.sync_copy(msg_hbm.at[pl.ds(wid * _EW, _EW), pl.ds(0, _AW)], msg_v)
    pltpu.sync_copy(zero_hbm.at[pl.ds(sid * _NSUB, _NSUB)],
                    acc_sh.at[pl.ds(sid * _NSUB, _NSUB)])
    plsc.subcore_barrier()
    for j in range(_KC):
        pltpu.sync_copy(msg_v.at[pl.ds(j * 128, 128)],
                        acc_sh.at[idx_v.at[j]], add=True)
    plsc.subcore_barrier()
    off = cid * _NP + sid * _NSUB
    pltpu.sync_copy(acc_sh.at[pl.ds(sid * _NSUB, _NSUB)],
                    out_hbm.at[pl.ds(off, _NSUB), pl.ds(0, _AW)])


_SCATTER_FN = None


def _sc_scatter_add(msg, idx3, zero128):
    global _SCATTER_FN
    if _SCATTER_FN is None:
        _SCATTER_FN = pl.kernel(
            _scatter_body,
            mesh=_sc_mesh(),
            out_type=jax.ShapeDtypeStruct((2 * _NP, _MW), jnp.float32),
            scratch_types=[
                pltpu.VMEM((8, 128), jnp.int32),
                pltpu.VMEM((_EW, _AW), jnp.float32),
                pltpu.VMEM_SHARED((_NP, _AW), jnp.float32),
            ],
            compiler_params=pltpu.CompilerParams(use_tc_tiling_on_sc=False),
        )
    return _SCATTER_FN(msg, idx3, zero128)


def _dense_body(in_ch, ea_ref, wa_ref, ba_ref, wb_ref, bb_ref, hs_ref, out_ref):
    b = pl.program_id(0)
    eb = ea_ref.shape[0]
    h = wa_ref.shape[1]
    f32 = jnp.float32
    a = jnp.dot(ea_ref[...], wa_ref[...], preferred_element_type=f32) + ba_ref[...]
    r = jnp.maximum(a, 0.0)
    w = jnp.dot(r, wb_ref[...], preferred_element_type=f32) + bb_ref[...]
    kcol = lax.broadcasted_iota(jnp.int32, (in_ch, h), 1) // _HID
    irow = lax.broadcasted_iota(jnp.int32, (in_ch, h), 0)
    rmat = (kcol == irow).astype(f32)
    hmat = jnp.dot(hs_ref[:, :in_ch], rmat, preferred_element_type=f32)
    krow = lax.broadcasted_iota(jnp.int32, (h, _HID), 0) % _HID
    ocol = lax.broadcasted_iota(jnp.int32, (h, _HID), 1)
    smat = (krow == ocol).astype(f32)
    msg = jnp.dot(hmat * w, smat, preferred_element_type=f32)
    rows = b * eb + lax.broadcasted_iota(jnp.int32, (eb, 1), 0)
    valid = rows < _E
    out_ref[...] = jnp.concatenate(
        [jnp.where(valid, msg, 0.0), valid.astype(f32),
         jnp.zeros((eb, _MW - _HID - 1), f32)], axis=1)


def _tc_dense(ea, wa, ba, wb, bb, hs, in_ch):
    h = wa.shape[1]
    grid = (_EP // _EB,)
    return pl.pallas_call(
        functools.partial(_dense_body, in_ch),
        grid=grid,
        in_specs=[
            pl.BlockSpec((_EB, _EDGE_IN), lambda i: (i, 0)),
            pl.BlockSpec((_EDGE_IN, h), lambda i: (0, 0)),
            pl.BlockSpec((1, h), lambda i: (0, 0)),
            pl.BlockSpec((h, h), lambda i: (0, 0)),
            pl.BlockSpec((1, h), lambda i: (0, 0)),
            pl.BlockSpec((_EB, _MW), lambda i: (i, 0)),
        ],
        out_specs=pl.BlockSpec((_EB, _MW), lambda i: (i, 0)),
        out_shape=jax.ShapeDtypeStruct((_EP, _MW), jnp.float32),
    )(ea, wa, ba.reshape(1, h), wb, bb.reshape(1, h), hs)


def _agg_bn_relu(acc, hprev, root, bias, gamma, beta):
    f32 = jnp.float32
    sp = acc[0] + acc[1]
    s = sp[:, :_HID]
    cnt = sp[:, _HID:_HID + 1]
    inv = 1.0 / jnp.maximum(cnt, 1.0)
    pre = s * inv + jnp.dot(hprev, root, preferred_element_type=f32) + bias
    rows = lax.broadcasted_iota(jnp.int32, (pre.shape[0], 1), 0)
    mask = rows < _N
    pre_m = jnp.where(mask, pre, 0.0)
    mu = jnp.sum(pre_m, axis=0, keepdims=True) * (1.0 / _N)
    var = jnp.sum(jnp.where(mask, (pre - mu) ** 2, 0.0), axis=0,
                  keepdims=True) * (1.0 / _N)
    hn = gamma * (pre - mu) * lax.rsqrt(var + 1e-5) + beta
    return jnp.where(mask, jnp.maximum(hn, 0.0), 0.0)


def _combine_body(acc_ref, x_ref, root_ref, bias_ref, gamma_ref, beta_ref,
                  out_ref):
    h = _agg_bn_relu(acc_ref[...], x_ref[...], root_ref[...],
                     bias_ref[...], gamma_ref[...], beta_ref[...])
    out_ref[...] = jnp.concatenate(
        [h, jnp.zeros((h.shape[0], _MW - _HID), jnp.float32)], axis=1)


def _tc_combine(acc, hprev, root, bias, gamma, beta):
    nin = hprev.shape[1]
    return pl.pallas_call(
        _combine_body,
        grid=(1,),
        in_specs=[
            pl.BlockSpec((2, _NP, _MW), lambda i: (0, 0, 0)),
            pl.BlockSpec((_NP, nin), lambda i: (0, 0)),
            pl.BlockSpec((nin, _HID), lambda i: (0, 0)),
            pl.BlockSpec((1, _HID), lambda i: (0, 0)),
            pl.BlockSpec((1, _HID), lambda i: (0, 0)),
            pl.BlockSpec((1, _HID), lambda i: (0, 0)),
        ],
        out_specs=pl.BlockSpec((_NP, _MW), lambda i: (0, 0)),
        out_shape=jax.ShapeDtypeStruct((_NP, _MW), jnp.float32),
    )(acc, hprev, root, bias.reshape(1, -1), gamma.reshape(1, -1),
      beta.reshape(1, -1))


def _final_body(acc_ref, h1_ref, root_ref, bias_ref, gamma_ref, beta_ref,
                batch_ref, wc1_ref, bc1_ref, wc2_ref, bc2_ref, out_ref):
    f32 = jnp.float32
    h2 = _agg_bn_relu(acc_ref[...], h1_ref[:, :_HID], root_ref[...],
                      bias_ref[...], gamma_ref[...], beta_ref[...])
    gid = lax.broadcasted_iota(jnp.int32, (_NP, _G), 1)
    onehot = (batch_ref[...] == gid).astype(f32)
    seg = lax.dot_general(onehot, h2, (((0,), (0,)), ((), ())),
                          preferred_element_type=f32)
    ones = jnp.ones((_NP, 1), f32)
    cg = lax.dot_general(onehot, ones, (((0,), (0,)), ((), ())),
                         preferred_element_type=f32)
    pooled = seg * (1.0 / jnp.maximum(cg, 1.0))
    hc = jnp.maximum(
        jnp.dot(pooled, wc1_ref[...], preferred_element_type=f32)
        + bc1_ref[...], 0.0)
    out_ref[...] = (jnp.dot(hc, wc2_ref[...], preferred_element_type=f32)
                    + bc2_ref[...])


def _tc_final(acc, h1, root, bias, gamma, beta, batch2, wc1, bc1, wc2, bc2):
    return pl.pallas_call(
        _final_body,
        grid=(1,),
        in_specs=[
            pl.BlockSpec((2, _NP, _MW), lambda i: (0, 0, 0)),
            pl.BlockSpec((_NP, _MW), lambda i: (0, 0)),
            pl.BlockSpec((_HID, _HID), lambda i: (0, 0)),
            pl.BlockSpec((1, _HID), lambda i: (0, 0)),
            pl.BlockSpec((1, _HID), lambda i: (0, 0)),
            pl.BlockSpec((1, _HID), lambda i: (0, 0)),
            pl.BlockSpec((_NP, 1), lambda i: (0, 0)),
            pl.BlockSpec((_HID, 32), lambda i: (0, 0)),
            pl.BlockSpec((1, 32), lambda i: (0, 0)),
            pl.BlockSpec((32, 1), lambda i: (0, 0)),
            pl.BlockSpec((1, 1), lambda i: (0, 0)),
        ],
        out_specs=pl.BlockSpec((_G, 1), lambda i: (0, 0)),
        out_shape=jax.ShapeDtypeStruct((_G, 1), jnp.float32),
    )(acc, h1, root, bias.reshape(1, -1), gamma.reshape(1, -1),
      beta.reshape(1, -1), batch2, wc1, bc1.reshape(1, -1), wc2,
      bc2.reshape(1, 1))


def _idx3(v, pad):
    v = jnp.concatenate([v, jnp.zeros((pad,), jnp.int32)]).reshape(_NW, _KC, 128)
    return jnp.pad(v, ((0, 0), (0, 8 - _KC), (0, 0)))


def kernel(x, edge_index, edge_attr, batch, W1a, b1a, W1b, b1b, root1, bias1,
           gamma1, beta1, W2a, b2a, W2b, b2b, root2, bias2, gamma2, beta2,
           Wc1, bc1, Wc2, bc2):
    f32 = jnp.float32
    i32 = jnp.int32
    pe = _EP - _E
    pn = _NP - _N
    src3 = _idx3(edge_index[0], pe)
    dst3 = _idx3(edge_index[1], pe)
    x128 = jnp.pad(x, ((0, pn), (0, _MW - _NODE_IN)))
    x_p = jnp.concatenate([x, jnp.zeros((pn, _NODE_IN), f32)], axis=0)
    batch2 = jnp.concatenate([batch, jnp.full((pn,), _G, i32)]).reshape(_NP, 1)
    zero128 = jnp.zeros((_NP, _AW), f32)

    xs = _sc_gather(x128, src3)
    m1 = _tc_dense(edge_attr, W1a, b1a, W1b, b1b, xs, _NODE_IN)
    a1 = _sc_scatter_add(m1, dst3, zero128)
    h1 = _tc_combine(a1.reshape(2, _NP, _MW), x_p, root1, bias1, gamma1, beta1)

    h1s = _sc_gather(h1, src3)
    m2 = _tc_dense(edge_attr, W2a, b2a, W2b, b2b, h1s, _HID)
    a2 = _sc_scatter_add(m2, dst3, zero128)

    return _tc_final(a2.reshape(2, _NP, _MW), h1, root2, bias2, gamma2, beta2,
                     batch2, Wc1, bc1, Wc2, bc2)
